# baseline (speedup 1.0000x reference)
import functools

import jax
import jax.numpy as jnp
from jax import lax
from jax.experimental import pallas as pl
from jax.experimental.pallas import tpu as pltpu
from jax.experimental.pallas import tpu_sc as plsc

ENT_N = 14541
REL_N = 237
DIM = 64
BATCH = 16384

NC = 2
NS = 16
NW = NC * NS
BPW = BATCH // NW
CHUNK = 128
NCHUNK = BPW // CHUNK

ROWS_W = 456
ENT_PAD = NW * ROWS_W
RELS_W = 8
RELPAD = 240
RTAB = RELPAD * 2

_INV1024 = 1.0 / 1024.0
_FEED = (0, 8, 4, 12, 2, 10, 6, 14, 1, 9, 5, 13, 3, 11, 7, 15)

_GDN = lax.GatherDimensionNumbers(
    offset_dims=(), collapsed_slice_dims=(0,), start_index_map=(0,))


def _take16(v, idx):
    return lax.gather(v, idx[:, None], _GDN, (1,),
                      mode=lax.GatherScatterMode.PROMISE_IN_BOUNDS)


def _tree_sum(vecs, perms, iota16):
    cur = list(vecs)
    for k, lanebit in ((3, 8), (2, 4), (1, 2), (0, 1)):
        sel = (iota16 & lanebit) == 0
        nxt = []
        for i in range(0, len(cur), 2):
            a, b = cur[i], cur[i + 1]
            sa = a + _take16(a, perms[k])
            sb = b + _take16(b, perms[k])
            nxt.append(jnp.where(sel, sa, sb))
        cur = nxt
    return cur[0]


def _rsqrt_nr(m):
    bits = lax.bitcast_convert_type(m, jnp.int32)
    y = lax.bitcast_convert_type(
        jnp.int32(0x5F3759DF) - lax.shift_right_logical(bits, 1),
        jnp.float32)
    for _ in range(3):
        y = y * (1.5 - 0.5 * m * y * y)
    return y


def _build_convert_kernel():
    mesh = plsc.VectorSubcoreMesh(core_axis_name="c", subcore_axis_name="s")

    @functools.partial(
        pl.kernel,
        mesh=mesh,
        out_type=(jax.ShapeDtypeStruct((ENT_PAD * DIM,), jnp.float32),
                  jax.ShapeDtypeStruct((RTAB * DIM,), jnp.float32)),
        compiler_params=pltpu.CompilerParams(
            use_tc_tiling_on_sc=True, needs_layout_passes=False),
        scratch_types=[
            pltpu.VMEM((ROWS_W, DIM), jnp.float32),
            pltpu.VMEM((RELS_W, 16, DIM), jnp.float32),
            pltpu.VMEM((ROWS_W * DIM,), jnp.float32),
            pltpu.VMEM((RELS_W * 2 * DIM,), jnp.float32),
            pltpu.SemaphoreType.DMA,
        ],
    )
    def convert_kernel(ent_hbm, rel_hbm, entout_hbm, relout_hbm, ein_v,
                       rin_v, eout_v, rout_v, sem0):
        wid = lax.axis_index("s") * NC + lax.axis_index("c")

        r0_ideal = wid * ROWS_W
        r0 = pl.multiple_of(
            jnp.minimum(r0_ideal, (ENT_N // 8 + 1) * 8 - ROWS_W), 8)
        roff = r0_ideal - r0
        cp_e = pltpu.async_copy(ent_hbm.at[pl.ds(r0, ROWS_W)], ein_v, sem0)

        rel0 = jnp.minimum(wid * RELS_W, RELPAD - RELS_W)
        rel_starts = []
        rel_cps = []
        for k in range(RELS_W):
            ri = jnp.minimum((rel0 + k) * (DIM + 1),
                             (REL_N - 1) * (DIM + 1))
            b0 = pl.multiple_of((ri // 8) * 8, 8)
            rel_starts.append(ri - b0)
            rel_cps.append(pltpu.async_copy(
                rel_hbm.at[pl.ds(b0, 16)], rin_v.at[k], sem0))
        cp_e.wait()
        for cp in rel_cps:
            cp.wait()

        def ebody(i, carry):
            src = jnp.minimum(i + roff, ROWS_W - 1)
            for s in range(DIM // 16):
                eout_v[pl.ds(i * DIM + s * 16, 16)] = (
                    ein_v[src, pl.ds(s * 16, 16)])
            return carry

        lax.fori_loop(0, ROWS_W, ebody, 0)
        for k in range(RELS_W):
            for b in range(2):
                row = rel_starts[k] + b
                for s in range(DIM // 16):
                    rout_v[pl.ds((2 * k + b) * DIM + s * 16, 16)] = (
                        rin_v[k, row, pl.ds(s * 16, 16)])

        pltpu.sync_copy(
            eout_v, entout_hbm.at[pl.ds(wid * (ROWS_W * DIM), ROWS_W * DIM)])
        pltpu.sync_copy(
            rout_v,
            relout_hbm.at[pl.ds(rel0 * 2 * DIM, RELS_W * 2 * DIM)])

    return convert_kernel


def _build_main_kernel():
    mesh = plsc.VectorSubcoreMesh(core_axis_name="c", subcore_axis_name="s")

    @functools.partial(
        pl.kernel,
        mesh=mesh,
        out_type=jax.ShapeDtypeStruct((BATCH,), jnp.float32),
        compiler_params=pltpu.CompilerParams(
            use_tc_tiling_on_sc=False, needs_layout_passes=False),
        scratch_types=[
            pltpu.VMEM((RTAB, DIM), jnp.float32),
            pltpu.VMEM((BPW,), jnp.int32),
            pltpu.VMEM((BPW,), jnp.int32),
            pltpu.VMEM((BPW, DIM), jnp.float32),
            pltpu.VMEM((BPW, DIM), jnp.float32),
            pltpu.VMEM((BPW,), jnp.int32),
            pltpu.VMEM((BPW,), jnp.float32),
            pltpu.SemaphoreType.DMA,
            pltpu.SemaphoreType.DMA,
            pltpu.SemaphoreType.DMA,
            pltpu.SemaphoreType.DMA,
        ],
    )
    def main_kernel(ent_hbm, relsub_hbm, hid_hbm, tid_hbm, rt_hbm, out_hbm,
                    reltab_v, idxh_v, idxt_v, hbuf_v, tbuf_v, rt_v, out_v,
                    sem0, sem1, sem2, sem3):
        wid = lax.axis_index("s") * NC + lax.axis_index("c")
        base = wid * BPW
        bsl = pl.ds(base, BPW)

        stage = [
            pltpu.async_copy(relsub_hbm, reltab_v, sem0),
            pltpu.async_copy(rt_hbm.at[bsl], rt_v, sem0),
            pltpu.async_copy(hid_hbm.at[bsl], idxh_v, sem0),
            pltpu.async_copy(tid_hbm.at[bsl], idxt_v, sem0),
        ]
        for cp in stage:
            cp.wait()

        csems = [sem0, sem1, sem2, sem3]
        gathers = []
        for c in range(NCHUNK):
            sl = pl.ds(c * CHUNK, CHUNK)
            gathers.append((
                pltpu.async_copy(
                    ent_hbm.at[idxh_v.at[sl]], hbuf_v.at[sl], csems[c]),
                pltpu.async_copy(
                    ent_hbm.at[idxt_v.at[sl]], tbuf_v.at[sl], csems[c])))

        iota16 = lax.iota(jnp.int32, 16)
        perms = [iota16 ^ (1 << k) for k in range(4)]

        def row_vecs(j, scale, rrow, row0):
            num = jnp.zeros((16,), jnp.float32)
            p2 = jnp.zeros((16,), jnp.float32)
            t2 = jnp.zeros((16,), jnp.float32)
            for s in range(DIM // 16):
                sl = pl.ds(s * 16, 16)
                hv = hbuf_v[row0 + j, sl]
                tv = tbuf_v[row0 + j, sl]
                rv = reltab_v[rrow, sl]
                ov = reltab_v[rrow + 1, sl]
                offs = scale * ov
                x = hv + rv - offs
                prod = x * x * _INV1024 + offs
                num = num + prod * tv
                p2 = p2 + prod * prod
                t2 = t2 + tv * tv
            return num, p2, t2

        def body(grp, carry):
            row0 = grp * 16
            gsl = pl.ds(row0, 16)
            rt16 = rt_v[gsl]
            scale16 = (rt16 + 1).astype(jnp.float32)
            nvecs, pvecs, tvecs = [], [], []
            for j in _FEED:
                num, p2, t2 = row_vecs(j, scale16[j], rt16[j] * 2, row0)
                nvecs.append(num)
                pvecs.append(p2)
                tvecs.append(t2)
            nsum = _tree_sum(nvecs, perms, iota16)
            psum = _tree_sum(pvecs, perms, iota16)
            tsum = _tree_sum(tvecs, perms, iota16)
            m = jnp.maximum(psum, 1e-16) * jnp.maximum(tsum, 1e-16)
            out_v[gsl] = -(nsum * _rsqrt_nr(m))
            return carry

        for c in range(NCHUNK):
            for cp in gathers[c]:
                cp.wait()
            lax.fori_loop(c * (CHUNK // 16), (c + 1) * (CHUNK // 16),
                          body, 0)
        pltpu.sync_copy(out_v, out_hbm.at[bsl])

    return main_kernel


_CONVERT_KERNEL = _build_convert_kernel()
_MAIN_KERNEL = _build_main_kernel()


def kernel(h_ids, r_typ, t_ids, ent_emb, rel_emb):
    ent_emb, rel_emb = lax.optimization_barrier((ent_emb, rel_emb))
    ent_lin, rel_lin = _CONVERT_KERNEL(ent_emb, rel_emb)
    return _MAIN_KERNEL(ent_lin.reshape(ENT_PAD, DIM),
                        rel_lin.reshape(RTAB, DIM),
                        h_ids.astype(jnp.int32), t_ids.astype(jnp.int32),
                        r_typ.astype(jnp.int32))


# speedup vs baseline: 1.0576x; 1.0576x over previous
import functools

import jax
import jax.numpy as jnp
from jax import lax
from jax.experimental import pallas as pl
from jax.experimental.pallas import tpu as pltpu
from jax.experimental.pallas import tpu_sc as plsc

ENT_N = 14541
REL_N = 237
DIM = 64
BATCH = 16384

NC = 2
NS = 16
NW = NC * NS
BPW = BATCH // NW
CHUNK = 128
NCHUNK = BPW // CHUNK
RTAB = 2 * REL_N

_INV1024 = 1.0 / 1024.0
_FEED = (0, 8, 4, 12, 2, 10, 6, 14, 1, 9, 5, 13, 3, 11, 7, 15)

_GDN = lax.GatherDimensionNumbers(
    offset_dims=(), collapsed_slice_dims=(0,), start_index_map=(0,))


def _take16(v, idx):
    return lax.gather(v, idx[:, None], _GDN, (1,),
                      mode=lax.GatherScatterMode.PROMISE_IN_BOUNDS)


def _tree_sum(vecs, perms, iota16):
    cur = list(vecs)
    for k, lanebit in ((3, 8), (2, 4), (1, 2), (0, 1)):
        sel = (iota16 & lanebit) == 0
        nxt = []
        for i in range(0, len(cur), 2):
            a, b = cur[i], cur[i + 1]
            sa = a + _take16(a, perms[k])
            sb = b + _take16(b, perms[k])
            nxt.append(jnp.where(sel, sa, sb))
        cur = nxt
    return cur[0]


def _rsqrt_nr(m):
    bits = lax.bitcast_convert_type(m, jnp.int32)
    y = lax.bitcast_convert_type(
        jnp.int32(0x5F3759DF) - lax.shift_right_logical(bits, 1),
        jnp.float32)
    for _ in range(3):
        y = y * (1.5 - 0.5 * m * y * y)
    return y


def _row_vecs(j, scale, rrow, reltab_v, hbuf_v, tbuf_v, row0):
    num = jnp.zeros((16,), jnp.float32)
    p2 = jnp.zeros((16,), jnp.float32)
    t2 = jnp.zeros((16,), jnp.float32)
    for s in range(DIM // 16):
        sl = pl.ds(s * 16, 16)
        hv = hbuf_v[row0 + j, sl]
        tv = tbuf_v[row0 + j, sl]
        rv = reltab_v[rrow, sl]
        ov = reltab_v[rrow + 1, sl]
        offs = scale * ov
        x = hv + rv - offs
        prod = x * x * _INV1024 + offs
        num = num + prod * tv
        p2 = p2 + prod * prod
        t2 = t2 + tv * tv
    return num, p2, t2


def _build_sc_kernel():
    mesh = plsc.VectorSubcoreMesh(core_axis_name="c", subcore_axis_name="s")

    @functools.partial(
        pl.kernel,
        mesh=mesh,
        out_type=jax.ShapeDtypeStruct((BATCH,), jnp.float32),
        compiler_params=pltpu.CompilerParams(
            use_tc_tiling_on_sc=False, needs_layout_passes=False),
        scratch_types=[
            pltpu.VMEM((RTAB, DIM), jnp.float32),
            pltpu.VMEM((BPW,), jnp.int32),
            pltpu.VMEM((BPW,), jnp.int32),
            pltpu.VMEM((BPW, DIM), jnp.float32),
            pltpu.VMEM((BPW, DIM), jnp.float32),
            pltpu.VMEM((BPW,), jnp.int32),
            pltpu.VMEM((BPW,), jnp.float32),
            pltpu.SemaphoreType.DMA,
        ],
    )
    def sc_kernel(ent_hbm, relsub_hbm, hid_hbm, tid_hbm, rt_hbm, out_hbm,
                  reltab_v, idxh_v, idxt_v, hbuf_v, tbuf_v, rt_v, out_v,
                  sem0):
        wid = lax.axis_index("s") * NC + lax.axis_index("c")
        base = wid * BPW
        bsl = pl.ds(base, BPW)

        stage = [
            pltpu.async_copy(relsub_hbm, reltab_v, sem0),
            pltpu.async_copy(rt_hbm.at[bsl], rt_v, sem0),
            pltpu.async_copy(hid_hbm.at[bsl], idxh_v, sem0),
            pltpu.async_copy(tid_hbm.at[bsl], idxt_v, sem0),
        ]
        for cp in stage:
            cp.wait()

        gathers = []
        for c in range(NCHUNK):
            sl = pl.ds(c * CHUNK, CHUNK)
            gathers.append(pltpu.async_copy(
                ent_hbm.at[idxh_v.at[sl]], hbuf_v.at[sl], sem0))
            gathers.append(pltpu.async_copy(
                ent_hbm.at[idxt_v.at[sl]], tbuf_v.at[sl], sem0))
        for cp in gathers:
            cp.wait()

        iota16 = lax.iota(jnp.int32, 16)
        perms = [iota16 ^ (1 << k) for k in range(4)]

        def body(grp, carry):
            row0 = grp * 16
            gsl = pl.ds(row0, 16)
            rt16 = rt_v[gsl]
            scale16 = (rt16 + 1).astype(jnp.float32)
            nvecs, pvecs, tvecs = [], [], []
            for j in _FEED:
                num, p2, t2 = _row_vecs(j, scale16[j], rt16[j] * 2,
                                        reltab_v, hbuf_v, tbuf_v, row0)
                nvecs.append(num)
                pvecs.append(p2)
                tvecs.append(t2)
            nsum = _tree_sum(nvecs, perms, iota16)
            psum = _tree_sum(pvecs, perms, iota16)
            tsum = _tree_sum(tvecs, perms, iota16)
            m = jnp.maximum(psum, 1e-16) * jnp.maximum(tsum, 1e-16)
            out_v[gsl] = -(nsum * _rsqrt_nr(m))
            return carry

        lax.fori_loop(0, BPW // 16, body, 0)
        pltpu.sync_copy(out_v, out_hbm.at[bsl])

    return sc_kernel


_SC_KERNEL = _build_sc_kernel()


def kernel(h_ids, r_typ, t_ids, ent_emb, rel_emb):
    rel_sub = rel_emb.reshape(REL_N, DIM + 1, DIM)[:, :2, :]
    rel_sub = rel_sub.reshape(RTAB, DIM)
    return _SC_KERNEL(ent_emb, rel_sub, h_ids.astype(jnp.int32),
                      t_ids.astype(jnp.int32), r_typ.astype(jnp.int32))


# speedup vs baseline: 1.1173x; 1.0564x over previous
import functools

import jax
import jax.numpy as jnp
from jax import lax
from jax.experimental import pallas as pl
from jax.experimental.pallas import tpu as pltpu
from jax.experimental.pallas import tpu_sc as plsc

ENT_N = 14541
REL_N = 237
DIM = 64
BATCH = 16384

NC = 2
NS = 16
NW = NC * NS
BPW = BATCH // NW
CHUNK = 128
NCHUNK = BPW // CHUNK
RTAB = 2 * REL_N

_INV1024 = 1.0 / 1024.0
_FEED = (0, 8, 4, 12, 2, 10, 6, 14, 1, 9, 5, 13, 3, 11, 7, 15)

_GDN = lax.GatherDimensionNumbers(
    offset_dims=(), collapsed_slice_dims=(0,), start_index_map=(0,))


def _take16(v, idx):
    return lax.gather(v, idx[:, None], _GDN, (1,),
                      mode=lax.GatherScatterMode.PROMISE_IN_BOUNDS)


def _tree_sum(vecs, perms, iota16):
    cur = list(vecs)
    for k, lanebit in ((3, 8), (2, 4), (1, 2), (0, 1)):
        sel = (iota16 & lanebit) == 0
        nxt = []
        for i in range(0, len(cur), 2):
            a, b = cur[i], cur[i + 1]
            sa = a + _take16(a, perms[k])
            sb = b + _take16(b, perms[k])
            nxt.append(jnp.where(sel, sa, sb))
        cur = nxt
    return cur[0]


def _rsqrt_nr(m):
    bits = lax.bitcast_convert_type(m, jnp.int32)
    y = lax.bitcast_convert_type(
        jnp.int32(0x5F3759DF) - lax.shift_right_logical(bits, 1),
        jnp.float32)
    for _ in range(3):
        y = y * (1.5 - 0.5 * m * y * y)
    return y


def _row_vecs(j, scale, rrow, reltab_v, hbuf_v, tbuf_v, row0):
    num = jnp.zeros((16,), jnp.float32)
    p2 = jnp.zeros((16,), jnp.float32)
    t2 = jnp.zeros((16,), jnp.float32)
    for s in range(DIM // 16):
        sl = pl.ds(s * 16, 16)
        hv = hbuf_v[row0 + j, sl]
        tv = tbuf_v[row0 + j, sl]
        rv = reltab_v[rrow, sl]
        ov = reltab_v[rrow + 1, sl]
        offs = scale * ov
        x = hv + rv - offs
        prod = x * x * _INV1024 + offs
        num = num + prod * tv
        p2 = p2 + prod * prod
        t2 = t2 + tv * tv
    return num, p2, t2


def _build_sc_kernel():
    mesh = plsc.VectorSubcoreMesh(core_axis_name="c", subcore_axis_name="s")

    @functools.partial(
        pl.kernel,
        mesh=mesh,
        out_type=jax.ShapeDtypeStruct((BATCH,), jnp.float32),
        compiler_params=pltpu.CompilerParams(
            use_tc_tiling_on_sc=False, needs_layout_passes=False),
        scratch_types=[
            pltpu.VMEM((RTAB, DIM), jnp.float32),
            pltpu.VMEM((BPW,), jnp.int32),
            pltpu.VMEM((BPW,), jnp.int32),
            pltpu.VMEM((BPW, DIM), jnp.float32),
            pltpu.VMEM((BPW, DIM), jnp.float32),
            pltpu.VMEM((BPW,), jnp.int32),
            pltpu.VMEM((BPW,), jnp.float32),
            pltpu.SemaphoreType.DMA,
        ],
    )
    def sc_kernel(ent_hbm, relsub_hbm, hid_hbm, tid_hbm, rt_hbm, out_hbm,
                  reltab_v, idxh_v, idxt_v, hbuf_v, tbuf_v, rt_v, out_v,
                  sem0):
        wid = lax.axis_index("s") * NC + lax.axis_index("c")
        base = wid * BPW
        bsl = pl.ds(base, BPW)

        stage = [
            pltpu.async_copy(relsub_hbm, reltab_v, sem0),
            pltpu.async_copy(rt_hbm.at[bsl], rt_v, sem0),
            pltpu.async_copy(hid_hbm.at[bsl], idxh_v, sem0),
            pltpu.async_copy(tid_hbm.at[bsl], idxt_v, sem0),
        ]
        for cp in stage:
            cp.wait()

        gathers = []
        for c in range(NCHUNK):
            sl = pl.ds(c * CHUNK, CHUNK)
            gathers.append(pltpu.async_copy(
                ent_hbm.at[idxh_v.at[sl]], hbuf_v.at[sl], sem0))
            gathers.append(pltpu.async_copy(
                ent_hbm.at[idxt_v.at[sl]], tbuf_v.at[sl], sem0))
        for cp in gathers:
            cp.wait()

        iota16 = lax.iota(jnp.int32, 16)
        perms = [iota16 ^ (1 << k) for k in range(4)]

        def body(grp, carry):
            row0 = grp * 16
            gsl = pl.ds(row0, 16)
            rt16 = rt_v[gsl]
            scale16 = (rt16 + 1).astype(jnp.float32)
            nvecs, pvecs, tvecs = [], [], []
            for j in _FEED:
                num, p2, t2 = _row_vecs(j, scale16[j], rt16[j] * 2,
                                        reltab_v, hbuf_v, tbuf_v, row0)
                nvecs.append(num)
                pvecs.append(p2)
                tvecs.append(t2)
            nsum = _tree_sum(nvecs, perms, iota16)
            psum = _tree_sum(pvecs, perms, iota16)
            tsum = _tree_sum(tvecs, perms, iota16)
            m = jnp.maximum(psum, 1e-16) * jnp.maximum(tsum, 1e-16)
            out_v[gsl] = -(nsum * _rsqrt_nr(m))
            return carry

        lax.fori_loop(0, BPW // 16, body, 0)
        pltpu.sync_copy(out_v, out_hbm.at[bsl])

    return sc_kernel


_SC_KERNEL = _build_sc_kernel()


def kernel(h_ids, r_typ, t_ids, ent_emb, rel_emb):
    i = jnp.arange(RTAB, dtype=jnp.int32)
    sub_rows = (i >> 1) * (DIM + 1) + (i & 1)
    rel_sub = jnp.take(rel_emb, sub_rows, axis=0)
    return _SC_KERNEL(ent_emb, rel_sub, h_ids.astype(jnp.int32),
                      t_ids.astype(jnp.int32), r_typ.astype(jnp.int32))
